# baseline (speedup 1.0000x reference)
import functools

import jax
import jax.numpy as jnp
from jax import lax
from jax.experimental import pallas as pl
from jax.experimental.pallas import tpu as pltpu
from jax.experimental.pallas import tpu_sc as plsc

N = 10000
E = 320000
F_IN = 128
D = 128
H1 = 6
NB = 128

_ROWS = 2000


def _k1_body(x_ref, w_ref, asrc_ref, adst_ref, h_ref, aS_ref, aD_ref):
    h = jnp.dot(x_ref[...], w_ref[...], preferred_element_type=jnp.float32)
    h_ref[...] = h
    h3 = h.reshape(_ROWS, H1, D)
    a_s = jnp.sum(h3 * asrc_ref[...][None], axis=-1)
    a_d = jnp.sum(h3 * adst_ref[...][None], axis=-1)
    pad = jnp.zeros((_ROWS, 16 - H1), jnp.float32)
    aS_ref[...] = jnp.concatenate([a_s, pad], axis=1)
    aD_ref[...] = jnp.concatenate([a_d, pad], axis=1)


def _k1(x, W1, a_src1, a_dst1):
    grid = N // _ROWS
    return pl.pallas_call(
        _k1_body,
        grid=(grid,),
        in_specs=[
            pl.BlockSpec((_ROWS, F_IN), lambda i: (i, 0)),
            pl.BlockSpec((F_IN, H1 * D), lambda i: (0, 0)),
            pl.BlockSpec((H1, D), lambda i: (0, 0)),
            pl.BlockSpec((H1, D), lambda i: (0, 0)),
        ],
        out_specs=[
            pl.BlockSpec((_ROWS, H1 * D), lambda i: (i, 0)),
            pl.BlockSpec((_ROWS, 16), lambda i: (i, 0)),
            pl.BlockSpec((_ROWS, 16), lambda i: (i, 0)),
        ],
        out_shape=[
            jax.ShapeDtypeStruct((N, H1 * D), jnp.float32),
            jax.ShapeDtypeStruct((N, 16), jnp.float32),
            jax.ShapeDtypeStruct((N, 16), jnp.float32),
        ],
    )(x, W1, a_src1, a_dst1)


def _k4_body(acc_ref, dena_ref, denb_ref, b1_ref, w2_ref, asrc_ref, adst_ref,
             h2_ref, aS2_ref, aD2_ref):
    acc = acc_ref[...].reshape(_ROWS, H1, D)
    den = (dena_ref[...] + denb_ref[...])[:, :H1]
    x1 = acc / (den[:, :, None] + 1e-16)
    x1 = jnp.maximum(x1.reshape(_ROWS, H1 * D) + b1_ref[...][None, :], 0.0)
    h2 = jnp.dot(x1, w2_ref[...], preferred_element_type=jnp.float32)
    h2_ref[...] = h2
    a_s = jnp.sum(h2 * asrc_ref[...][0][None, :], axis=-1, keepdims=True)
    a_d = jnp.sum(h2 * adst_ref[...][0][None, :], axis=-1, keepdims=True)
    pad = jnp.zeros((_ROWS, 15), jnp.float32)
    aS2_ref[...] = jnp.concatenate([a_s, pad], axis=1)
    aD2_ref[...] = jnp.concatenate([a_d, pad], axis=1)


def _k4(acc1, den1a, den1b, b1, W2, a_src2, a_dst2):
    grid = N // _ROWS
    return pl.pallas_call(
        _k4_body,
        grid=(grid,),
        in_specs=[
            pl.BlockSpec((_ROWS, H1 * D), lambda i: (i, 0)),
            pl.BlockSpec((_ROWS, 16), lambda i: (i, 0)),
            pl.BlockSpec((_ROWS, 16), lambda i: (i, 0)),
            pl.BlockSpec((H1 * D,), lambda i: (0,)),
            pl.BlockSpec((H1 * D, D), lambda i: (0, 0)),
            pl.BlockSpec((1, D), lambda i: (0, 0)),
            pl.BlockSpec((1, D), lambda i: (0, 0)),
        ],
        out_specs=[
            pl.BlockSpec((_ROWS, D), lambda i: (i, 0)),
            pl.BlockSpec((_ROWS, 16), lambda i: (i, 0)),
            pl.BlockSpec((_ROWS, 16), lambda i: (i, 0)),
        ],
        out_shape=[
            jax.ShapeDtypeStruct((N, D), jnp.float32),
            jax.ShapeDtypeStruct((N, 16), jnp.float32),
            jax.ShapeDtypeStruct((N, 16), jnp.float32),
        ],
    )(acc1, den1a, den1b, b1, W2, a_src2, a_dst2)


def _k7_body(acca_ref, accb_ref, dena_ref, denb_ref, b2_ref, batch_ref,
             drug2_ref, wfc_ref, bfc_ref, wfin_ref, bfin_ref, out_ref,
             pool_ref):
    i = pl.program_id(0)

    @pl.when(i == 0)
    def _init():
        pool_ref[...] = jnp.zeros_like(pool_ref)

    biota = lax.broadcasted_iota(jnp.int32, (NB, 8), 0)
    b2row = b2_ref[...][None, :]

    def chunk(c, pool):
        acc8 = acca_ref[pl.ds(c * 8, 8), :] + accb_ref[pl.ds(c * 8, 8), :]
        den8 = (dena_ref[pl.ds(c * 8, 8), :]
                + denb_ref[pl.ds(c * 8, 8), :])[:, :1]
        rows = jnp.maximum(acc8 / (den8 + 1e-16) + b2row, 0.0)
        ids = batch_ref[0, pl.ds(c * 8, 8), 0]
        mt = biota == ids[None, :]
        for i in range(8):
            pool = jnp.where(mt[:, i:i + 1],
                             jnp.maximum(pool, rows[i:i + 1, :]), pool)
        return pool

    pool_ref[...] = lax.fori_loop(0, _ROWS // 8, chunk, pool_ref[...])

    @pl.when(i == pl.num_programs(0) - 1)
    def _tail():
        p = pool_ref[...]
        x3 = jnp.maximum(
            jnp.dot(p, wfc_ref[...], preferred_element_type=jnp.float32)
            + bfc_ref[...][None, :], 0.0)
        ff = (x3 + drug2_ref[...]) * 0.5
        out_ref[...] = (
            jnp.dot(ff, wfin_ref[...], preferred_element_type=jnp.float32)
            + bfin_ref[...][None, :])


def _k7(acc2a, acc2b, den2a, den2b, b2, batch, drug2, W_fc, b_fc, W_final,
        b_final):
    grid = N // _ROWS
    nout = W_final.shape[1]
    return pl.pallas_call(
        _k7_body,
        grid=(grid,),
        in_specs=[
            pl.BlockSpec((_ROWS, D), lambda i: (i, 0)),
            pl.BlockSpec((_ROWS, D), lambda i: (i, 0)),
            pl.BlockSpec((_ROWS, 16), lambda i: (i, 0)),
            pl.BlockSpec((_ROWS, 16), lambda i: (i, 0)),
            pl.BlockSpec((D,), lambda i: (0,)),
            pl.BlockSpec((1, _ROWS, 1), lambda i: (i, 0, 0)),
            pl.BlockSpec((NB, D), lambda i: (0, 0)),
            pl.BlockSpec((D, D), lambda i: (0, 0)),
            pl.BlockSpec((D,), lambda i: (0,)),
            pl.BlockSpec((D, nout), lambda i: (0, 0)),
            pl.BlockSpec((nout,), lambda i: (0,)),
        ],
        out_specs=pl.BlockSpec((NB, nout), lambda i: (0, 0)),
        out_shape=jax.ShapeDtypeStruct((NB, nout), jnp.float32),
        scratch_shapes=[pltpu.VMEM((NB, D), jnp.float32)],
    )(acc2a, acc2b, den2a, den2b, b2, batch.reshape(grid, _ROWS, 1), drug2,
      W_fc, b_fc, W_final, b_final)


_NC, _NS, _NW = 2, 16, 32
_EW = E // _NW
_C2 = 400
_SUB = 80
_NSUB = _C2 // _SUB
_NCH2 = _EW // _C2
_NSL = 624


def _edge_softmax(src_arr, dst_arr, alphS, alphD, z16):
    mesh = plsc.VectorSubcoreMesh(core_axis_name="c", subcore_axis_name="s")

    @functools.partial(
        pl.kernel, mesh=mesh,
        compiler_params=pltpu.CompilerParams(use_tc_tiling_on_sc=False, needs_layout_passes=False),
        out_type=[
            jax.ShapeDtypeStruct((E, 16), jnp.float32),
            jax.ShapeDtypeStruct((N, 16), jnp.float32),
            jax.ShapeDtypeStruct((N, 16), jnp.float32),
        ],
        scratch_types=[
            pltpu.VMEM((_C2,), jnp.int32),
            pltpu.VMEM((_NSUB, _SUB), jnp.int32),
            pltpu.VMEM((_C2, 16), jnp.float32),
            pltpu.VMEM((_C2, 16), jnp.float32),
            pltpu.VMEM((_C2, 16), jnp.float32),
            pltpu.VMEM_SHARED((N, 16), jnp.float32),
            pltpu.SemaphoreType.DMA,
        ],
    )
    def k(src_hbm, dst_hbm, alphS_hbm, alphD_hbm, z16_hbm, ex_hbm, den0_hbm,
          den1_hbm, src_v, dst_v, as_v, ad_v, ex_v, den_sh, sem):
        cid = lax.axis_index("c")
        sid = lax.axis_index("s")
        wid = cid * _NS + sid
        row0 = sid * _NSL
        pltpu.sync_copy(z16_hbm.at[pl.ds(row0, _NSL)],
                        den_sh.at[pl.ds(row0, _NSL)])

        @pl.when(sid == _NS - 1)
        def _ztail():
            pltpu.sync_copy(z16_hbm.at[pl.ds(_NS * _NSL, N - _NS * _NSL)],
                            den_sh.at[pl.ds(_NS * _NSL, N - _NS * _NSL)])

        plsc.subcore_barrier()

        @pl.loop(0, _NCH2)
        def _chunk(ci):
            base = wid * _EW + ci * _C2
            pltpu.sync_copy(src_hbm.at[pl.ds(base, _C2)], src_v)
            for j in range(_NSUB):
                pltpu.sync_copy(dst_hbm.at[pl.ds(base + j * _SUB, _SUB)],
                                dst_v.at[j])
            cps = []
            for j in range(_NSUB):
                sl = pl.ds(j * _SUB, _SUB)
                cps.append(pltpu.async_copy(
                    alphS_hbm.at[src_v.at[sl]], as_v.at[sl, :], sem))
                cps.append(pltpu.async_copy(
                    alphD_hbm.at[dst_v.at[j]], ad_v.at[sl, :], sem))
            for cp in cps:
                cp.wait()

            @pl.loop(0, _C2)
            def _edge(e):
                v = as_v[e, :] + ad_v[e, :]
                v = jnp.where(v > 0, v, 0.2 * v)
                ex_v[e, :] = jnp.exp(v)

            pltpu.sync_copy(ex_v, ex_hbm.at[pl.ds(base, _C2), :])
            for j in range(_NSUB):
                pltpu.sync_copy(ex_v.at[pl.ds(j * _SUB, _SUB), :],
                                den_sh.at[dst_v.at[j]], add=True)

        plsc.subcore_barrier()

        tail0, tailn = _NS * _NSL, N - _NS * _NSL

        @pl.when(cid == 0)
        def _out0():
            pltpu.sync_copy(den_sh.at[pl.ds(row0, _NSL)],
                            den0_hbm.at[pl.ds(row0, _NSL)])

            @pl.when(sid == _NS - 1)
            def _t0():
                pltpu.sync_copy(den_sh.at[pl.ds(tail0, tailn)],
                                den0_hbm.at[pl.ds(tail0, tailn)])

        @pl.when(cid == 1)
        def _out1():
            pltpu.sync_copy(den_sh.at[pl.ds(row0, _NSL)],
                            den1_hbm.at[pl.ds(row0, _NSL)])

            @pl.when(sid == _NS - 1)
            def _t1():
                pltpu.sync_copy(den_sh.at[pl.ds(tail0, tailn)],
                                den1_hbm.at[pl.ds(tail0, tailn)])

    return k(src_arr, dst_arr, alphS, alphD, z16)


_C6 = 400
_NSB6 = _C6 // 16


def _agg2(src_arr, dst_arr, ex2, h2, z128):
    mesh = plsc.VectorSubcoreMesh(core_axis_name="c", subcore_axis_name="s")

    @functools.partial(
        pl.kernel, mesh=mesh,
        compiler_params=pltpu.CompilerParams(use_tc_tiling_on_sc=False, needs_layout_passes=False),
        out_type=[
            jax.ShapeDtypeStruct((N, D), jnp.float32),
            jax.ShapeDtypeStruct((N, D), jnp.float32),
        ],
        scratch_types=[
            pltpu.VMEM((_C6,), jnp.int32),
            pltpu.VMEM((_C6,), jnp.int32),
            pltpu.VMEM((16,), jnp.int32),
            pltpu.VMEM((_C6, 16), jnp.float32),
            pltpu.VMEM((2, 16, D), jnp.float32),
            pltpu.VMEM_SHARED((N, D), jnp.float32),
            pltpu.SemaphoreType.DMA,
            pltpu.SemaphoreType.DMA,
        ],
    )
    def k(src_hbm, dst_hbm, ex_hbm, h_hbm, z_hbm, acc0_hbm, acc1_hbm,
          src_v, dst_v, drow_v, ex_v, h_v, acc_sh, sem_a, sem_b):
        cid = lax.axis_index("c")
        sid = lax.axis_index("s")
        wid = cid * _NS + sid
        row0 = sid * _NSL
        pltpu.sync_copy(z_hbm.at[pl.ds(row0, _NSL)],
                        acc_sh.at[pl.ds(row0, _NSL)])

        @pl.when(sid == _NS - 1)
        def _ztail():
            pltpu.sync_copy(z_hbm.at[pl.ds(_NS * _NSL, N - _NS * _NSL)],
                            acc_sh.at[pl.ds(_NS * _NSL, N - _NS * _NSL)])

        plsc.subcore_barrier()

        @pl.loop(0, _EW // _C6)
        def _chunk(ci):
            base = wid * _EW + ci * _C6
            pltpu.sync_copy(src_hbm.at[pl.ds(base, _C6)], src_v)
            pltpu.sync_copy(dst_hbm.at[pl.ds(base, _C6)], dst_v)
            pltpu.sync_copy(ex_hbm.at[pl.ds(base, _C6), :], ex_v)
            sems = (sem_a, sem_b)

            def _fire(s, par):
                pltpu.async_copy(h_hbm.at[src_v.at[pl.ds(s * 16, 16)]],
                                 h_v.at[par], sems[par])

            _fire(jnp.int32(0), 0)

            @pl.loop(0, _NSB6)
            def _sub(s):
                par = s % 2
                for q in range(2):
                    @pl.when((s + 1 < _NSB6) & ((s + 1) % 2 == q))
                    def _next():
                        _fire(s + 1, q)

                    @pl.when(par == q)
                    def _wait():
                        pltpu.make_async_copy(
                            h_hbm.at[src_v.at[pl.ds(s * 16, 16)]],
                            h_v.at[q], sems[q]).wait()

                for i in range(16):
                    coeff = plsc.load_gather(
                        ex_v, [jnp.full((16,), s * 16 + i, jnp.int32),
                               jnp.zeros((16,), jnp.int32)])
                    for v in range(D // 16):
                        vsl = pl.ds(v * 16, 16)
                        h_v[par, i, vsl] = h_v[par, i, vsl] * coeff
                drow_v[...] = dst_v[pl.ds(s * 16, 16)]
                pltpu.sync_copy(h_v.at[par], acc_sh.at[drow_v], add=True)

        plsc.subcore_barrier()
        tail0, tailn = _NS * _NSL, N - _NS * _NSL

        @pl.when(cid == 0)
        def _out0():
            pltpu.sync_copy(acc_sh.at[pl.ds(row0, _NSL)],
                            acc0_hbm.at[pl.ds(row0, _NSL)])

            @pl.when(sid == _NS - 1)
            def _t0():
                pltpu.sync_copy(acc_sh.at[pl.ds(tail0, tailn)],
                                acc0_hbm.at[pl.ds(tail0, tailn)])

        @pl.when(cid == 1)
        def _out1():
            pltpu.sync_copy(acc_sh.at[pl.ds(row0, _NSL)],
                            acc1_hbm.at[pl.ds(row0, _NSL)])

            @pl.when(sid == _NS - 1)
            def _t1():
                pltpu.sync_copy(acc_sh.at[pl.ds(tail0, tailn)],
                                acc1_hbm.at[pl.ds(tail0, tailn)])

    return k(src_arr, dst_arr, ex2, h2, z128)


_R3 = 1672
_SP3 = 1680
_TRASH3 = 1676
_NP3 = 3
_C3 = 2000
_ES = E // _NS
_HD = H1 * D


def _agg1(src_arr, dst_arr, ex1, h1, z768):
    mesh = plsc.VectorSubcoreMesh(core_axis_name="c", subcore_axis_name="s")

    @functools.partial(
        pl.kernel, mesh=mesh,
        compiler_params=pltpu.CompilerParams(use_tc_tiling_on_sc=False,
                                             needs_layout_passes=False),
        out_type=jax.ShapeDtypeStruct((N, _HD), jnp.float32),
        scratch_types=[
            pltpu.VMEM((_C3,), jnp.int32),
            pltpu.VMEM((_C3,), jnp.int32),
            pltpu.VMEM((_C3 + 16,), jnp.int32),
            pltpu.VMEM((_C3 + 16,), jnp.int32),
            pltpu.VMEM((_C3 + 16,), jnp.int32),
            pltpu.VMEM((16,), jnp.int32),
            pltpu.VMEM((2, 16, 16), jnp.float32),
            pltpu.VMEM((2, 16, _HD), jnp.float32),
            pltpu.VMEM_SHARED((_SP3, _HD), jnp.float32),
            pltpu.SemaphoreType.DMA,
            pltpu.SemaphoreType.DMA,
        ],
    )
    def k(src_hbm, dst_hbm, ex_hbm, h_hbm, z_hbm, acc_hbm,
          src_v, dst_v, csrc_v, ceid_v, cdr_v, drow_v, exg_v, hg_v,
          acc_sh, sem_a, sem_b):
        cid = lax.axis_index("c")
        sid = lax.axis_index("s")
        iota16 = jnp.arange(16, dtype=jnp.int32)

        for p in range(_NP3):
            rid = cid * _NP3 + p
            lo = rid * _R3
            nz = _SP3 // _NS
            pltpu.sync_copy(z_hbm.at[pl.ds(sid * nz, nz)],
                            acc_sh.at[pl.ds(sid * nz, nz)])
            plsc.subcore_barrier()

            @pl.loop(0, _ES // _C3)
            def _chunk(ci):
                cbase = sid * _ES + ci * _C3
                pltpu.sync_copy(src_hbm.at[pl.ds(cbase, _C3)], src_v)
                pltpu.sync_copy(dst_hbm.at[pl.ds(cbase, _C3)], dst_v)

                def _scan(g, ptr):
                    sl = pl.ds(g * 16, 16)
                    d16 = dst_v[sl]
                    s16 = src_v[sl]
                    eid16 = iota16 + (cbase + g * 16)
                    dr = d16 - lo
                    m = (dr >= 0) & (dr < _R3)
                    psl = pl.ds(ptr, 16)
                    plsc.store_compressed(csrc_v.at[psl], s16, mask=m)
                    plsc.store_compressed(ceid_v.at[psl], eid16, mask=m)
                    plsc.store_compressed(cdr_v.at[psl], dr, mask=m)
                    return ptr + jnp.sum(m.astype(jnp.int32))

                n = lax.fori_loop(0, _C3 // 16, _scan, jnp.int32(0))
                pad = pl.ds(n, 16)
                csrc_v[pad] = jnp.zeros((16,), jnp.int32)
                ceid_v[pad] = jnp.zeros((16,), jnp.int32)
                cdr_v[pad] = jnp.full((16,), _TRASH3, jnp.int32)
                nsb = (n + 15) // 16
                sems = (sem_a, sem_b)

                def _fire(s, par):
                    sl = pl.ds(s * 16, 16)
                    pltpu.async_copy(ex_hbm.at[ceid_v.at[sl]],
                                     exg_v.at[par], sems[par])
                    pltpu.async_copy(h_hbm.at[csrc_v.at[sl]],
                                     hg_v.at[par], sems[par])

                @pl.when(nsb > 0)
                def _prime():
                    _fire(jnp.int32(0), 0)

                @pl.loop(0, nsb)
                def _sub(s):
                    par = s % 2
                    for q in range(2):
                        @pl.when((s + 1 < nsb) & ((s + 1) % 2 == q))
                        def _next():
                            _fire(s + 1, q)

                        @pl.when(par == q)
                        def _wait():
                            sl = pl.ds(s * 16, 16)
                            pltpu.make_async_copy(
                                ex_hbm.at[ceid_v.at[sl]], exg_v.at[q],
                                sems[q]).wait()
                            pltpu.make_async_copy(
                                h_hbm.at[csrc_v.at[sl]], hg_v.at[q],
                                sems[q]).wait()

                    for i in range(16):
                        for hd in range(H1):
                            coeff = plsc.load_gather(
                                exg_v.at[par],
                                [jnp.full((16,), i, jnp.int32),
                                 jnp.full((16,), hd, jnp.int32)])
                            for v in range(D // 16):
                                vsl = pl.ds(hd * D + v * 16, 16)
                                hg_v[par, i, vsl] = hg_v[par, i, vsl] * coeff

                    drow_v[...] = cdr_v[pl.ds(s * 16, 16)]
                    pltpu.sync_copy(hg_v.at[par], acc_sh.at[drow_v], add=True)

            plsc.subcore_barrier()

            @pl.when((sid == 0) & (rid < 5))
            def _out():
                pltpu.sync_copy(acc_sh.at[pl.ds(0, _R3)],
                                acc_hbm.at[pl.ds(lo, _R3)])

            @pl.when((sid == 0) & (rid == 5))
            def _outlast():
                pltpu.sync_copy(acc_sh.at[pl.ds(0, N - 5 * _R3)],
                                acc_hbm.at[pl.ds(5 * _R3, N - 5 * _R3)])

            plsc.subcore_barrier()

    return k(src_arr, dst_arr, ex1, h1, z768)


def _agg_xla(h, ex, src, dst, heads):
    h3 = h.reshape(N, heads, D)
    exh = ex[:, :heads]
    acc = jax.ops.segment_sum(h3[src] * exh[:, :, None], dst, num_segments=N)
    return acc.reshape(N, heads * D)


def kernel(x, edge_index, batch, drug2, W1, a_src1, a_dst1, b1, W2, a_src2,
           a_dst2, b2, W_fc, b_fc, W_final, b_final):
    src, dst = edge_index[0], edge_index[1]
    z16 = jnp.zeros((N, 16), jnp.float32)
    h1, aS1, aD1 = _k1(x, W1, a_src1, a_dst1)
    ex1, den1a, den1b = _edge_softmax(src, dst, aS1, aD1, z16)
    z768 = jnp.zeros((_SP3, _HD), jnp.float32)
    acc1 = _agg1(src, dst, ex1, h1, z768)
    h2, aS2, aD2 = _k4(acc1, den1a, den1b, b1, W2, a_src2, a_dst2)
    ex2, den2a, den2b = _edge_softmax(src, dst, aS2, aD2, z16)
    z128 = jnp.zeros((N, D), jnp.float32)
    acc2a, acc2b = _agg2(src, dst, ex2, h2, z128)
    return _k7(acc2a, acc2b, den2a, den2b, b2, batch, drug2, W_fc, b_fc,
               W_final, b_final)


# speedup vs baseline: 1.0750x; 1.0750x over previous
import functools

import jax
import jax.numpy as jnp
from jax import lax
from jax.experimental import pallas as pl
from jax.experimental.pallas import tpu as pltpu
from jax.experimental.pallas import tpu_sc as plsc

N = 10000
E = 320000
F_IN = 128
D = 128
H1 = 6
NB = 128

_ROWS = 2000


def _k1_body(x_ref, w_ref, asrc_ref, adst_ref, h_ref, aS_ref, aD_ref):
    h = jnp.dot(x_ref[...], w_ref[...], preferred_element_type=jnp.float32)
    h_ref[...] = h
    h3 = h.reshape(_ROWS, H1, D)
    a_s = jnp.sum(h3 * asrc_ref[...][None], axis=-1)
    a_d = jnp.sum(h3 * adst_ref[...][None], axis=-1)
    pad = jnp.zeros((_ROWS, 16 - H1), jnp.float32)
    aS_ref[...] = jnp.concatenate([a_s, pad], axis=1)
    aD_ref[...] = jnp.concatenate([a_d, pad], axis=1)


def _k1(x, W1, a_src1, a_dst1):
    grid = N // _ROWS
    return pl.pallas_call(
        _k1_body,
        grid=(grid,),
        in_specs=[
            pl.BlockSpec((_ROWS, F_IN), lambda i: (i, 0)),
            pl.BlockSpec((F_IN, H1 * D), lambda i: (0, 0)),
            pl.BlockSpec((H1, D), lambda i: (0, 0)),
            pl.BlockSpec((H1, D), lambda i: (0, 0)),
        ],
        out_specs=[
            pl.BlockSpec((_ROWS, H1 * D), lambda i: (i, 0)),
            pl.BlockSpec((_ROWS, 16), lambda i: (i, 0)),
            pl.BlockSpec((_ROWS, 16), lambda i: (i, 0)),
        ],
        out_shape=[
            jax.ShapeDtypeStruct((N, H1 * D), jnp.float32),
            jax.ShapeDtypeStruct((N, 16), jnp.float32),
            jax.ShapeDtypeStruct((N, 16), jnp.float32),
        ],
    )(x, W1, a_src1, a_dst1)


def _k4_body(acc_ref, dena_ref, denb_ref, b1_ref, w2_ref, asrc_ref, adst_ref,
             h2_ref, aS2_ref, aD2_ref):
    acc = acc_ref[...].reshape(_ROWS, H1, D)
    den = (dena_ref[...] + denb_ref[...])[:, :H1]
    x1 = acc / (den[:, :, None] + 1e-16)
    x1 = jnp.maximum(x1.reshape(_ROWS, H1 * D) + b1_ref[...][None, :], 0.0)
    h2 = jnp.dot(x1, w2_ref[...], preferred_element_type=jnp.float32)
    h2_ref[...] = h2
    a_s = jnp.sum(h2 * asrc_ref[...][0][None, :], axis=-1, keepdims=True)
    a_d = jnp.sum(h2 * adst_ref[...][0][None, :], axis=-1, keepdims=True)
    pad = jnp.zeros((_ROWS, 15), jnp.float32)
    aS2_ref[...] = jnp.concatenate([a_s, pad], axis=1)
    aD2_ref[...] = jnp.concatenate([a_d, pad], axis=1)


def _k4(acc1, den1a, den1b, b1, W2, a_src2, a_dst2):
    grid = N // _ROWS
    return pl.pallas_call(
        _k4_body,
        grid=(grid,),
        in_specs=[
            pl.BlockSpec((_ROWS, H1 * D), lambda i: (i, 0)),
            pl.BlockSpec((_ROWS, 16), lambda i: (i, 0)),
            pl.BlockSpec((_ROWS, 16), lambda i: (i, 0)),
            pl.BlockSpec((H1 * D,), lambda i: (0,)),
            pl.BlockSpec((H1 * D, D), lambda i: (0, 0)),
            pl.BlockSpec((1, D), lambda i: (0, 0)),
            pl.BlockSpec((1, D), lambda i: (0, 0)),
        ],
        out_specs=[
            pl.BlockSpec((_ROWS, D), lambda i: (i, 0)),
            pl.BlockSpec((_ROWS, 16), lambda i: (i, 0)),
            pl.BlockSpec((_ROWS, 16), lambda i: (i, 0)),
        ],
        out_shape=[
            jax.ShapeDtypeStruct((N, D), jnp.float32),
            jax.ShapeDtypeStruct((N, 16), jnp.float32),
            jax.ShapeDtypeStruct((N, 16), jnp.float32),
        ],
    )(acc1, den1a, den1b, b1, W2, a_src2, a_dst2)


def _k7_body(acca_ref, accb_ref, dena_ref, denb_ref, b2_ref, batch_ref,
             drug2_ref, wfc_ref, bfc_ref, wfin_ref, bfin_ref, out_ref,
             pool_ref):
    i = pl.program_id(0)

    @pl.when(i == 0)
    def _init():
        pool_ref[...] = jnp.zeros_like(pool_ref)

    biota = lax.broadcasted_iota(jnp.int32, (NB, 8), 0)
    b2row = b2_ref[...][None, :]

    def chunk(c, pool):
        acc8 = acca_ref[pl.ds(c * 8, 8), :] + accb_ref[pl.ds(c * 8, 8), :]
        den8 = (dena_ref[pl.ds(c * 8, 8), :]
                + denb_ref[pl.ds(c * 8, 8), :])[:, :1]
        rows = jnp.maximum(acc8 / (den8 + 1e-16) + b2row, 0.0)
        ids = batch_ref[0, pl.ds(c * 8, 8), 0]
        mt = biota == ids[None, :]
        for i in range(8):
            pool = jnp.where(mt[:, i:i + 1],
                             jnp.maximum(pool, rows[i:i + 1, :]), pool)
        return pool

    pool_ref[...] = lax.fori_loop(0, _ROWS // 8, chunk, pool_ref[...])

    @pl.when(i == pl.num_programs(0) - 1)
    def _tail():
        p = pool_ref[...]
        x3 = jnp.maximum(
            jnp.dot(p, wfc_ref[...], preferred_element_type=jnp.float32)
            + bfc_ref[...][None, :], 0.0)
        ff = (x3 + drug2_ref[...]) * 0.5
        out_ref[...] = (
            jnp.dot(ff, wfin_ref[...], preferred_element_type=jnp.float32)
            + bfin_ref[...][None, :])


def _k7(acc2a, acc2b, den2a, den2b, b2, batch, drug2, W_fc, b_fc, W_final,
        b_final):
    grid = N // _ROWS
    nout = W_final.shape[1]
    return pl.pallas_call(
        _k7_body,
        grid=(grid,),
        in_specs=[
            pl.BlockSpec((_ROWS, D), lambda i: (i, 0)),
            pl.BlockSpec((_ROWS, D), lambda i: (i, 0)),
            pl.BlockSpec((_ROWS, 16), lambda i: (i, 0)),
            pl.BlockSpec((_ROWS, 16), lambda i: (i, 0)),
            pl.BlockSpec((D,), lambda i: (0,)),
            pl.BlockSpec((1, _ROWS, 1), lambda i: (i, 0, 0)),
            pl.BlockSpec((NB, D), lambda i: (0, 0)),
            pl.BlockSpec((D, D), lambda i: (0, 0)),
            pl.BlockSpec((D,), lambda i: (0,)),
            pl.BlockSpec((D, nout), lambda i: (0, 0)),
            pl.BlockSpec((nout,), lambda i: (0,)),
        ],
        out_specs=pl.BlockSpec((NB, nout), lambda i: (0, 0)),
        out_shape=jax.ShapeDtypeStruct((NB, nout), jnp.float32),
        scratch_shapes=[pltpu.VMEM((NB, D), jnp.float32)],
    )(acc2a, acc2b, den2a, den2b, b2, batch.reshape(grid, _ROWS, 1), drug2,
      W_fc, b_fc, W_final, b_final)


_NC, _NS, _NW = 2, 16, 32
_EW = E // _NW
_C2 = 400
_SUB = 80
_NSUB = _C2 // _SUB
_NCH2 = _EW // _C2
_NSL = 624


def _edge_softmax(src_arr, dst_arr, alphS, alphD, z16):
    mesh = plsc.VectorSubcoreMesh(core_axis_name="c", subcore_axis_name="s")

    @functools.partial(
        pl.kernel, mesh=mesh,
        compiler_params=pltpu.CompilerParams(use_tc_tiling_on_sc=False, needs_layout_passes=False),
        out_type=[
            jax.ShapeDtypeStruct((E, 16), jnp.float32),
            jax.ShapeDtypeStruct((N, 16), jnp.float32),
            jax.ShapeDtypeStruct((N, 16), jnp.float32),
        ],
        scratch_types=[
            pltpu.VMEM((_C2,), jnp.int32),
            pltpu.VMEM((_NSUB, _SUB), jnp.int32),
            pltpu.VMEM((_C2, 16), jnp.float32),
            pltpu.VMEM((_C2, 16), jnp.float32),
            pltpu.VMEM((_C2, 16), jnp.float32),
            pltpu.VMEM_SHARED((N, 16), jnp.float32),
            pltpu.SemaphoreType.DMA,
        ],
    )
    def k(src_hbm, dst_hbm, alphS_hbm, alphD_hbm, z16_hbm, ex_hbm, den0_hbm,
          den1_hbm, src_v, dst_v, as_v, ad_v, ex_v, den_sh, sem):
        cid = lax.axis_index("c")
        sid = lax.axis_index("s")
        wid = cid * _NS + sid
        row0 = sid * _NSL
        pltpu.sync_copy(z16_hbm.at[pl.ds(row0, _NSL)],
                        den_sh.at[pl.ds(row0, _NSL)])

        @pl.when(sid == _NS - 1)
        def _ztail():
            pltpu.sync_copy(z16_hbm.at[pl.ds(_NS * _NSL, N - _NS * _NSL)],
                            den_sh.at[pl.ds(_NS * _NSL, N - _NS * _NSL)])

        plsc.subcore_barrier()

        @pl.loop(0, _NCH2)
        def _chunk(ci):
            base = wid * _EW + ci * _C2
            cps = [pltpu.async_copy(src_hbm.at[pl.ds(base, _C2)], src_v, sem)]
            for j in range(_NSUB):
                cps.append(pltpu.async_copy(
                    dst_hbm.at[pl.ds(base + j * _SUB, _SUB)], dst_v.at[j],
                    sem))
            for cp in cps:
                cp.wait()
            cps = []
            for j in range(_NSUB):
                sl = pl.ds(j * _SUB, _SUB)
                cps.append(pltpu.async_copy(
                    alphS_hbm.at[src_v.at[sl]], as_v.at[sl, :], sem))
                cps.append(pltpu.async_copy(
                    alphD_hbm.at[dst_v.at[j]], ad_v.at[sl, :], sem))
            for cp in cps:
                cp.wait()

            @pl.loop(0, _C2)
            def _edge(e):
                v = as_v[e, :] + ad_v[e, :]
                v = jnp.where(v > 0, v, 0.2 * v)
                ex_v[e, :] = jnp.exp(v)

            pltpu.sync_copy(ex_v, ex_hbm.at[pl.ds(base, _C2), :])
            for j in range(_NSUB):
                pltpu.sync_copy(ex_v.at[pl.ds(j * _SUB, _SUB), :],
                                den_sh.at[dst_v.at[j]], add=True)

        plsc.subcore_barrier()

        tail0, tailn = _NS * _NSL, N - _NS * _NSL

        @pl.when(cid == 0)
        def _out0():
            pltpu.sync_copy(den_sh.at[pl.ds(row0, _NSL)],
                            den0_hbm.at[pl.ds(row0, _NSL)])

            @pl.when(sid == _NS - 1)
            def _t0():
                pltpu.sync_copy(den_sh.at[pl.ds(tail0, tailn)],
                                den0_hbm.at[pl.ds(tail0, tailn)])

        @pl.when(cid == 1)
        def _out1():
            pltpu.sync_copy(den_sh.at[pl.ds(row0, _NSL)],
                            den1_hbm.at[pl.ds(row0, _NSL)])

            @pl.when(sid == _NS - 1)
            def _t1():
                pltpu.sync_copy(den_sh.at[pl.ds(tail0, tailn)],
                                den1_hbm.at[pl.ds(tail0, tailn)])

    return k(src_arr, dst_arr, alphS, alphD, z16)


_C6 = 400
_NSB6 = _C6 // 16


def _agg2(src_arr, dst_arr, ex2, h2, z128):
    mesh = plsc.VectorSubcoreMesh(core_axis_name="c", subcore_axis_name="s")

    @functools.partial(
        pl.kernel, mesh=mesh,
        compiler_params=pltpu.CompilerParams(use_tc_tiling_on_sc=False, needs_layout_passes=False),
        out_type=[
            jax.ShapeDtypeStruct((N, D), jnp.float32),
            jax.ShapeDtypeStruct((N, D), jnp.float32),
        ],
        scratch_types=[
            pltpu.VMEM((_C6,), jnp.int32),
            pltpu.VMEM((_C6,), jnp.int32),
            pltpu.VMEM((16,), jnp.int32),
            pltpu.VMEM((_C6, 16), jnp.float32),
            pltpu.VMEM((2, 16, D), jnp.float32),
            pltpu.VMEM_SHARED((N, D), jnp.float32),
            pltpu.SemaphoreType.DMA,
            pltpu.SemaphoreType.DMA,
        ],
    )
    def k(src_hbm, dst_hbm, ex_hbm, h_hbm, z_hbm, acc0_hbm, acc1_hbm,
          src_v, dst_v, drow_v, ex_v, h_v, acc_sh, sem_a, sem_b):
        cid = lax.axis_index("c")
        sid = lax.axis_index("s")
        wid = cid * _NS + sid
        row0 = sid * _NSL
        pltpu.sync_copy(z_hbm.at[pl.ds(row0, _NSL)],
                        acc_sh.at[pl.ds(row0, _NSL)])

        @pl.when(sid == _NS - 1)
        def _ztail():
            pltpu.sync_copy(z_hbm.at[pl.ds(_NS * _NSL, N - _NS * _NSL)],
                            acc_sh.at[pl.ds(_NS * _NSL, N - _NS * _NSL)])

        plsc.subcore_barrier()

        @pl.loop(0, _EW // _C6)
        def _chunk(ci):
            base = wid * _EW + ci * _C6
            pltpu.sync_copy(src_hbm.at[pl.ds(base, _C6)], src_v)
            pltpu.sync_copy(dst_hbm.at[pl.ds(base, _C6)], dst_v)
            pltpu.sync_copy(ex_hbm.at[pl.ds(base, _C6), :], ex_v)
            sems = (sem_a, sem_b)

            def _fire(s, par):
                pltpu.async_copy(h_hbm.at[src_v.at[pl.ds(s * 16, 16)]],
                                 h_v.at[par], sems[par])

            _fire(jnp.int32(0), 0)

            @pl.loop(0, _NSB6)
            def _sub(s):
                par = s % 2
                for q in range(2):
                    @pl.when((s + 1 < _NSB6) & ((s + 1) % 2 == q))
                    def _next():
                        _fire(s + 1, q)

                    @pl.when(par == q)
                    def _wait():
                        pltpu.make_async_copy(
                            h_hbm.at[src_v.at[pl.ds(s * 16, 16)]],
                            h_v.at[q], sems[q]).wait()

                for i in range(16):
                    coeff = plsc.load_gather(
                        ex_v, [jnp.full((16,), s * 16 + i, jnp.int32),
                               jnp.zeros((16,), jnp.int32)])
                    for v in range(D // 16):
                        vsl = pl.ds(v * 16, 16)
                        h_v[par, i, vsl] = h_v[par, i, vsl] * coeff
                drow_v[...] = dst_v[pl.ds(s * 16, 16)]
                pltpu.sync_copy(h_v.at[par], acc_sh.at[drow_v], add=True)

        plsc.subcore_barrier()
        tail0, tailn = _NS * _NSL, N - _NS * _NSL

        @pl.when(cid == 0)
        def _out0():
            pltpu.sync_copy(acc_sh.at[pl.ds(row0, _NSL)],
                            acc0_hbm.at[pl.ds(row0, _NSL)])

            @pl.when(sid == _NS - 1)
            def _t0():
                pltpu.sync_copy(acc_sh.at[pl.ds(tail0, tailn)],
                                acc0_hbm.at[pl.ds(tail0, tailn)])

        @pl.when(cid == 1)
        def _out1():
            pltpu.sync_copy(acc_sh.at[pl.ds(row0, _NSL)],
                            acc1_hbm.at[pl.ds(row0, _NSL)])

            @pl.when(sid == _NS - 1)
            def _t1():
                pltpu.sync_copy(acc_sh.at[pl.ds(tail0, tailn)],
                                acc1_hbm.at[pl.ds(tail0, tailn)])

    return k(src_arr, dst_arr, ex2, h2, z128)


_R3 = 1672
_SP3 = 1680
_TRASH3 = 1676
_NP3 = 3
_C3 = 2000
_ES = E // _NS
_HD = H1 * D


def _agg1(src_arr, dst_arr, ex1, h1, z768):
    mesh = plsc.VectorSubcoreMesh(core_axis_name="c", subcore_axis_name="s")

    @functools.partial(
        pl.kernel, mesh=mesh,
        compiler_params=pltpu.CompilerParams(use_tc_tiling_on_sc=False,
                                             needs_layout_passes=False),
        out_type=jax.ShapeDtypeStruct((N, _HD), jnp.float32),
        scratch_types=[
            pltpu.VMEM((_C3,), jnp.int32),
            pltpu.VMEM((_C3,), jnp.int32),
            pltpu.VMEM((_C3 + 16,), jnp.int32),
            pltpu.VMEM((_C3 + 16,), jnp.int32),
            pltpu.VMEM((_C3 + 16,), jnp.int32),
            pltpu.VMEM((16,), jnp.int32),
            pltpu.VMEM((2, 16, 16), jnp.float32),
            pltpu.VMEM((2, 16, _HD), jnp.float32),
            pltpu.VMEM_SHARED((_SP3, _HD), jnp.float32),
            pltpu.SemaphoreType.DMA,
            pltpu.SemaphoreType.DMA,
        ],
    )
    def k(src_hbm, dst_hbm, ex_hbm, h_hbm, z_hbm, acc_hbm,
          src_v, dst_v, csrc_v, ceid_v, cdr_v, drow_v, exg_v, hg_v,
          acc_sh, sem_a, sem_b):
        cid = lax.axis_index("c")
        sid = lax.axis_index("s")
        iota16 = jnp.arange(16, dtype=jnp.int32)

        for p in range(_NP3):
            rid = cid * _NP3 + p
            lo = rid * _R3
            nz = _SP3 // _NS
            pltpu.sync_copy(z_hbm.at[pl.ds(sid * nz, nz)],
                            acc_sh.at[pl.ds(sid * nz, nz)])
            plsc.subcore_barrier()

            @pl.loop(0, _ES // _C3)
            def _chunk(ci):
                cbase = sid * _ES + ci * _C3
                pltpu.sync_copy(src_hbm.at[pl.ds(cbase, _C3)], src_v)
                pltpu.sync_copy(dst_hbm.at[pl.ds(cbase, _C3)], dst_v)

                def _scan(g, ptr):
                    sl = pl.ds(g * 16, 16)
                    d16 = dst_v[sl]
                    s16 = src_v[sl]
                    eid16 = iota16 + (cbase + g * 16)
                    dr = d16 - lo
                    m = (dr >= 0) & (dr < _R3)
                    psl = pl.ds(ptr, 16)
                    plsc.store_compressed(csrc_v.at[psl], s16, mask=m)
                    plsc.store_compressed(ceid_v.at[psl], eid16, mask=m)
                    plsc.store_compressed(cdr_v.at[psl], dr, mask=m)
                    return ptr + jnp.sum(m.astype(jnp.int32))

                n = lax.fori_loop(0, _C3 // 16, _scan, jnp.int32(0))
                pad = pl.ds(n, 16)
                csrc_v[pad] = jnp.zeros((16,), jnp.int32)
                ceid_v[pad] = jnp.zeros((16,), jnp.int32)
                cdr_v[pad] = jnp.full((16,), _TRASH3, jnp.int32)
                nsb = (n + 15) // 16
                sems = (sem_a, sem_b)

                def _fire(s, par):
                    sl = pl.ds(s * 16, 16)
                    pltpu.async_copy(ex_hbm.at[ceid_v.at[sl]],
                                     exg_v.at[par], sems[par])
                    pltpu.async_copy(h_hbm.at[csrc_v.at[sl]],
                                     hg_v.at[par], sems[par])

                @pl.when(nsb > 0)
                def _prime():
                    _fire(jnp.int32(0), 0)

                @pl.loop(0, nsb)
                def _sub(s):
                    par = s % 2
                    for q in range(2):
                        @pl.when((s + 1 < nsb) & ((s + 1) % 2 == q))
                        def _next():
                            _fire(s + 1, q)

                        @pl.when(par == q)
                        def _wait():
                            sl = pl.ds(s * 16, 16)
                            pltpu.make_async_copy(
                                ex_hbm.at[ceid_v.at[sl]], exg_v.at[q],
                                sems[q]).wait()
                            pltpu.make_async_copy(
                                h_hbm.at[csrc_v.at[sl]], hg_v.at[q],
                                sems[q]).wait()

                    @pl.loop(0, 16)
                    def _edge(i):
                        for hd in range(H1):
                            coeff = plsc.load_gather(
                                exg_v.at[par],
                                [jnp.full((16,), i, jnp.int32),
                                 jnp.full((16,), hd, jnp.int32)])
                            for v in range(D // 16):
                                vsl = pl.ds(hd * D + v * 16, 16)
                                hg_v[par, i, vsl] = hg_v[par, i, vsl] * coeff

                    drow_v[...] = cdr_v[pl.ds(s * 16, 16)]
                    pltpu.sync_copy(hg_v.at[par], acc_sh.at[drow_v], add=True)

            plsc.subcore_barrier()

            @pl.when((sid == 0) & (rid < 5))
            def _out():
                pltpu.sync_copy(acc_sh.at[pl.ds(0, _R3)],
                                acc_hbm.at[pl.ds(lo, _R3)])

            @pl.when((sid == 0) & (rid == 5))
            def _outlast():
                pltpu.sync_copy(acc_sh.at[pl.ds(0, N - 5 * _R3)],
                                acc_hbm.at[pl.ds(5 * _R3, N - 5 * _R3)])

            plsc.subcore_barrier()

    return k(src_arr, dst_arr, ex1, h1, z768)


def _agg_xla(h, ex, src, dst, heads):
    h3 = h.reshape(N, heads, D)
    exh = ex[:, :heads]
    acc = jax.ops.segment_sum(h3[src] * exh[:, :, None], dst, num_segments=N)
    return acc.reshape(N, heads * D)


def kernel(x, edge_index, batch, drug2, W1, a_src1, a_dst1, b1, W2, a_src2,
           a_dst2, b2, W_fc, b_fc, W_final, b_final):
    src, dst = edge_index[0], edge_index[1]
    z16 = jnp.zeros((N, 16), jnp.float32)
    h1, aS1, aD1 = _k1(x, W1, a_src1, a_dst1)
    ex1, den1a, den1b = _edge_softmax(src, dst, aS1, aD1, z16)
    z768 = jnp.zeros((_SP3, _HD), jnp.float32)
    acc1 = _agg1(src, dst, ex1, h1, z768)
    h2, aS2, aD2 = _k4(acc1, den1a, den1b, b1, W2, a_src2, a_dst2)
    ex2, den2a, den2b = _edge_softmax(src, dst, aS2, aD2, z16)
    z128 = jnp.zeros((N, D), jnp.float32)
    acc2a, acc2b = _agg2(src, dst, ex2, h2, z128)
    return _k7(acc2a, acc2b, den2a, den2b, b2, batch, drug2, W_fc, b_fc,
               W_final, b_final)
